# baseline (speedup 1.0000x reference)
import functools

import jax
import jax.numpy as jnp
from jax import lax
from jax.experimental import pallas as pl
from jax.experimental.pallas import tpu as pltpu
from jax.experimental.pallas import tpu_sc as plsc

_E = 8
_K = 2
_D = 1024
_B = 4096
_SWITCHLOSS = 0.01
_ZLOSS = 0.001
_LANES = 128

_BM_G = 512
_NB_G = _B // _BM_G
_A = 2 * _BM_G
_NA = _K * _B
_BLK = 256
_NPAD = _NA + _E * _BLK
_NBLK = _NPAD // _BLK

_NTILES = 32
_CHUNK = _NA // _NTILES
_SUB = 64
_NSUB = _CHUNK // _SUB


def _softmax_top2(x_blk, wgt, lanes):
    emask = lanes < _E
    logits = jnp.dot(x_blk, wgt, preferred_element_type=jnp.float32)
    logits = jnp.where(emask, logits, -1e30)
    m = jnp.max(logits, axis=1, keepdims=True)
    p = jnp.exp(logits - m)
    s = jnp.sum(p, axis=1, keepdims=True)
    probs = p / s
    lse = jnp.log(s) + m
    g1 = jnp.max(probs, axis=1, keepdims=True)
    i1 = jnp.min(jnp.where(probs == g1, lanes, _LANES), axis=1, keepdims=True)
    probs2 = jnp.where(lanes == i1, -1.0, probs)
    g2 = jnp.max(probs2, axis=1, keepdims=True)
    i2 = jnp.min(jnp.where(probs2 == g2, lanes, _LANES), axis=1, keepdims=True)
    return probs, lse, g1, i1, g2, i2


def _gating_body(x_ref, wgt_ref, g1_ref, g2_ref, slot_ref, bemap_ref,
                 loss_ref, psum_ref, freq_ref, cnt_ref, offs_ref, cnt2_ref,
                 zsum_ref):
    i = pl.program_id(0)
    lanes = jax.lax.broadcasted_iota(jnp.int32, (_BM_G, _LANES), 1)
    probs, lse, g1, i1, g2, i2 = _softmax_top2(x_ref[...], wgt_ref[...], lanes)
    oh1 = (lanes == i1).astype(jnp.float32)
    oh2 = (lanes == i2).astype(jnp.float32)
    cnt_blk = (jnp.sum(oh1, axis=0, keepdims=True)
               + jnp.sum(oh2, axis=0, keepdims=True))

    denom = g1 + g2 + 1e-6
    g1n = g1 / denom
    g2n = g2 / denom
    g1_ref[...] = g1n[None]
    g2_ref[...] = g2n[None]

    @pl.when(i < _NB_G)
    def _pass1():
        gz1 = jnp.where(lanes == i1, g1n, 0.0)
        gz2 = jnp.where(lanes == i2, g2n, 0.0)
        freq_blk = (jnp.sum((gz1 > 0.0).astype(jnp.float32), axis=0,
                            keepdims=True)
                    + jnp.sum((gz2 > 0.0).astype(jnp.float32), axis=0,
                              keepdims=True))
        psum_blk = jnp.sum(probs, axis=0, keepdims=True)
        z_blk = jnp.sum(lse * lse)

        @pl.when(i == 0)
        def _init():
            psum_ref[...] = psum_blk
            freq_ref[...] = freq_blk
            cnt_ref[...] = cnt_blk
            zsum_ref[0, 0] = z_blk

        @pl.when(i > 0)
        def _acc():
            psum_ref[...] += psum_blk
            freq_ref[...] += freq_blk
            cnt_ref[...] += cnt_blk
            zsum_ref[0, 0] += z_blk

        @pl.when(i == _NB_G - 1)
        def _finish():
            psum = psum_ref[...]
            pnorm = psum / jnp.sum(jnp.abs(psum))
            freqs = freq_ref[...]
            fnorm = freqs / jnp.sum(jnp.abs(freqs))
            switch = jnp.sum(pnorm * fnorm) * _E
            z = zsum_ref[0, 0] / _B
            loss = _SWITCHLOSS * switch + _ZLOSS * z
            loss_ref[...] = jnp.broadcast_to(loss, (1, _LANES))

            cnt_i = cnt_ref[...].astype(jnp.int32)
            padded = jnp.bitwise_and(cnt_i + (_BLK - 1), -_BLK)
            padded_f = padded.astype(jnp.float32)
            riota = jax.lax.broadcasted_iota(jnp.int32, (_LANES, _LANES), 0)
            ciota = jax.lax.broadcasted_iota(jnp.int32, (_LANES, _LANES), 1)
            upper = (riota < ciota).astype(jnp.float32)
            offs = jnp.dot(padded_f, upper,
                           preferred_element_type=jnp.float32)
            offs_ref[...] = offs
            ends = offs + padded_f
            lrow = jax.lax.broadcasted_iota(jnp.int32, (1, _LANES), 1)
            tstart = (lrow * _BLK).astype(jnp.float32)
            bemap = jnp.zeros((1, _LANES), jnp.float32)
            for e in range(_E):
                end_e = jnp.sum(jnp.where(lrow == e, ends, 0.0))
                bemap += (tstart >= end_e).astype(jnp.float32)
            bemap_ref[...] = jnp.minimum(bemap, float(_E - 1)).astype(jnp.int32)

    @pl.when(i >= _NB_G)
    def _pass2():
        o2 = jnp.concatenate([oh1, oh2], axis=0)
        riota = jax.lax.broadcasted_iota(jnp.int32, (_A, _A), 0)
        ciota = jax.lax.broadcasted_iota(jnp.int32, (_A, _A), 1)
        tril = (ciota < riota).astype(jnp.float32)
        ranks = jnp.dot(tril, o2, preferred_element_type=jnp.float32)

        @pl.when(i == _NB_G)
        def _init2():
            cnt2_ref[...] = jnp.zeros((1, _LANES), jnp.float32)

        slots = ranks + cnt2_ref[...] + offs_ref[...]
        slot_col = jnp.sum(slots * o2, axis=1, keepdims=True)
        slot_ref[...] = slot_col.astype(jnp.int32)[None]
        cnt2_ref[...] += cnt_blk


def _gating(x, wgt_pad, interpret=False):
    return pl.pallas_call(
        _gating_body,
        grid=(2 * _NB_G,),
        in_specs=[
            pl.BlockSpec((_BM_G, _D), lambda i: (lax.rem(i, _NB_G), 0)),
            pl.BlockSpec((_D, _LANES), lambda i: (0, 0)),
        ],
        out_specs=[
            pl.BlockSpec((1, _BM_G, 1), lambda i: (lax.rem(i, _NB_G), 0, 0)),
            pl.BlockSpec((1, _BM_G, 1), lambda i: (lax.rem(i, _NB_G), 0, 0)),
            pl.BlockSpec((1, _A, 1), lambda i: (lax.rem(i, _NB_G), 0, 0)),
            pl.BlockSpec((1, _LANES), lambda i: (0, 0)),
            pl.BlockSpec((1, _LANES), lambda i: (0, 0)),
        ],
        out_shape=[
            jax.ShapeDtypeStruct((_NB_G, _BM_G, 1), jnp.float32),
            jax.ShapeDtypeStruct((_NB_G, _BM_G, 1), jnp.float32),
            jax.ShapeDtypeStruct((_NB_G, _A, 1), jnp.int32),
            jax.ShapeDtypeStruct((1, _LANES), jnp.int32),
            jax.ShapeDtypeStruct((1, _LANES), jnp.float32),
        ],
        scratch_shapes=[
            pltpu.VMEM((1, _LANES), jnp.float32),
            pltpu.VMEM((1, _LANES), jnp.float32),
            pltpu.VMEM((1, _LANES), jnp.float32),
            pltpu.VMEM((1, _LANES), jnp.float32),
            pltpu.VMEM((1, _LANES), jnp.float32),
            pltpu.SMEM((1, 1), jnp.float32),
        ],
        interpret=interpret,
    )(x, wgt_pad)


def _dispatch_sc(x, tok, slot):
    mesh = plsc.VectorSubcoreMesh(core_axis_name="c", subcore_axis_name="s")

    @functools.partial(
        pl.kernel, mesh=mesh,
        out_type=jax.ShapeDtypeStruct((_NPAD, _D), jnp.float32),
        scratch_types=[
            pltpu.VMEM((_SUB,), jnp.int32),
            pltpu.VMEM((_SUB,), jnp.int32),
            pltpu.VMEM((_SUB, _D), jnp.float32),
            pltpu.SemaphoreType.DMA,
            pltpu.SemaphoreType.DMA,
        ],
    )
    def body(x_hbm, tok_hbm, slot_hbm, xs_hbm, idx_t, idx_s, rows, sem1,
             sem2):
        w = lax.axis_index("s") * 2 + lax.axis_index("c")
        for c in range(_NSUB):
            base = w * _CHUNK + c * _SUB
            pltpu.sync_copy(tok_hbm.at[pl.ds(base, _SUB)], idx_t)
            pltpu.sync_copy(slot_hbm.at[pl.ds(base, _SUB)], idx_s)
            pltpu.async_copy(x_hbm.at[idx_t], rows, sem1).wait()
            pltpu.async_copy(rows, xs_hbm.at[idx_s], sem2).wait()

    return body(x, tok, slot)


def _undispatch_sc(ys, slot):
    mesh = plsc.VectorSubcoreMesh(core_axis_name="c", subcore_axis_name="s")

    @functools.partial(
        pl.kernel, mesh=mesh,
        out_type=jax.ShapeDtypeStruct((_NA, _D), jnp.float32),
        scratch_types=[
            pltpu.VMEM((_SUB,), jnp.int32),
            pltpu.VMEM((_SUB, _D), jnp.float32),
            pltpu.SemaphoreType.DMA,
        ],
    )
    def body(ys_hbm, slot_hbm, yg_hbm, idx_s, rows, sem):
        w = lax.axis_index("s") * 2 + lax.axis_index("c")
        for c in range(_NSUB):
            base = w * _CHUNK + c * _SUB
            pltpu.sync_copy(slot_hbm.at[pl.ds(base, _SUB)], idx_s)
            pltpu.async_copy(ys_hbm.at[idx_s], rows, sem).wait()
            pltpu.sync_copy(rows, yg_hbm.at[pl.ds(base, _SUB)])

    return body(ys, slot)


def _grouped_body(bm_ref, x_ref, we_ref, be_ref, out_ref):
    out_ref[...] = (jnp.dot(x_ref[...], we_ref[0],
                            preferred_element_type=jnp.float32)
                    + be_ref[0])


def _grouped(xs, we, be3, bemap, interpret=False):
    grid_spec = pltpu.PrefetchScalarGridSpec(
        num_scalar_prefetch=1,
        grid=(_NBLK,),
        in_specs=[
            pl.BlockSpec((_BLK, _D), lambda t, m: (t, 0)),
            pl.BlockSpec((1, _D, _D), lambda t, m: (m[t], 0, 0)),
            pl.BlockSpec((1, 1, _D), lambda t, m: (m[t], 0, 0)),
        ],
        out_specs=pl.BlockSpec((_BLK, _D), lambda t, m: (t, 0)),
    )
    return pl.pallas_call(
        _grouped_body,
        grid_spec=grid_spec,
        out_shape=jax.ShapeDtypeStruct((_NPAD, _D), jnp.float32),
        interpret=interpret,
    )(bemap, xs, we, be3)


def _combine_body(y0_ref, y1_ref, g1_ref, g2_ref, out_ref):
    out_ref[...] = (g1_ref[0] * y0_ref[...] + g2_ref[0] * y1_ref[...])


def _combine(yg, g1, g2, interpret=False):
    return pl.pallas_call(
        _combine_body,
        grid=(_NB_G,),
        in_specs=[
            pl.BlockSpec((_BM_G, _D), lambda i: (2 * i, 0)),
            pl.BlockSpec((_BM_G, _D), lambda i: (2 * i + 1, 0)),
            pl.BlockSpec((1, _BM_G, 1), lambda i: (i, 0, 0)),
            pl.BlockSpec((1, _BM_G, 1), lambda i: (i, 0, 0)),
        ],
        out_specs=pl.BlockSpec((_BM_G, _D), lambda i: (i, 0)),
        out_shape=jax.ShapeDtypeStruct((_B, _D), jnp.float32),
        interpret=interpret,
    )(yg, yg, g1, g2)


@functools.partial(jax.jit, static_argnames=("interpret",))
def kernel(x, Wg, We, be, interpret=False):
    wgt_pad = jnp.zeros((_D, _LANES), jnp.float32).at[:, :_E].set(Wg.T)
    g1, g2, slot3, bemap_row, loss_row = _gating(x, wgt_pad,
                                                 interpret=interpret)
    slot = slot3.reshape(_NA)
    bemap = bemap_row.reshape(_LANES)
    j = jnp.arange(_NA, dtype=jnp.int32)
    tok = (j // _A) * _BM_G + (j % _BM_G)

    xs = _dispatch_sc(x, tok, slot)
    ys = _grouped(xs, We, be.reshape(_E, 1, _D), bemap, interpret=interpret)
    yg = _undispatch_sc(ys, slot)
    out = _combine(yg, g1, g2, interpret=interpret)
    return out, loss_row[0, 0]
